# baseline (speedup 1.0000x reference)
import functools

import jax
import jax.numpy as jnp
from jax.experimental import pallas as pl
from jax.experimental.pallas import tpu as pltpu


def _gate_kernel(x_ref, w1_ref, b1_ref, w2_ref,
                 scores_ref, idx_ref, loss_ref, acc_ref,
                 *, total_tokens: int, num_experts: int):
    i = pl.program_id(0)
    n = pl.num_programs(0)

    x = x_ref[...]
    h = jnp.maximum(
        jax.lax.dot_general(x, w1_ref[...], (((1,), (0,)), ((), ())),
                            preferred_element_type=jnp.float32,
                            precision=jax.lax.Precision.DEFAULT)
        + b1_ref[...],
        0.0)
    logits = jax.lax.dot_general(h, w2_ref[...], (((1,), (0,)), ((), ())),
                                 preferred_element_type=jnp.float32)

    row_max = jnp.max(logits, axis=-1, keepdims=True)
    ex = jnp.exp(logits - row_max)
    scores = ex / jnp.sum(ex, axis=-1, keepdims=True)

    iota = jax.lax.broadcasted_iota(jnp.int32, scores.shape, 1)
    m1 = jnp.max(scores, axis=-1, keepdims=True)
    a1 = jnp.min(jnp.where(scores == m1, iota, num_experts), axis=-1,
                 keepdims=True)
    masked = jnp.where(iota == a1, -jnp.inf, scores)
    m2 = jnp.max(masked, axis=-1, keepdims=True)
    a2 = jnp.min(jnp.where(masked == m2, iota, num_experts), axis=-1,
                 keepdims=True)

    t = jnp.exp(m2 - m1)
    denom = 1.0 + t
    scores_ref[...] = jnp.concatenate([1.0 / denom, t / denom], axis=-1)
    idx_ref[...] = jnp.concatenate([a1, a2], axis=-1)

    part = jnp.sum(scores, axis=0, keepdims=True)

    @pl.when(i == 0)
    def _():
        acc_ref[...] = part

    @pl.when(i > 0)
    def _():
        acc_ref[...] += part

    @pl.when(i == n - 1)
    def _():
        mean = acc_ref[...] / total_tokens
        loss_ref[...] = num_experts * jnp.sum(mean * jnp.log(mean + 1e-8),
                                              axis=-1, keepdims=True)


def kernel(x, W1, b1, W2):
    batch, seq, d_model = x.shape
    m = batch * seq
    d_hidden = W1.shape[1]
    num_experts = W2.shape[1]

    tm = min(512, m)
    grid = (m // tm,)

    x_flat = x.reshape(m, d_model)
    b1_2d = b1.reshape(1, d_hidden)
    w1_b = W1.astype(jnp.bfloat16)

    scores, idx, loss = pl.pallas_call(
        functools.partial(_gate_kernel, total_tokens=m,
                          num_experts=num_experts),
        grid=grid,
        in_specs=[
            pl.BlockSpec((tm, d_model), lambda i: (i, 0)),
            pl.BlockSpec((d_model, d_hidden), lambda i: (0, 0)),
            pl.BlockSpec((1, d_hidden), lambda i: (0, 0)),
            pl.BlockSpec((d_hidden, num_experts), lambda i: (0, 0)),
        ],
        out_specs=[
            pl.BlockSpec((tm, 2), lambda i: (i, 0)),
            pl.BlockSpec((tm, 2), lambda i: (i, 0)),
            pl.BlockSpec((1, 1), lambda i: (0, 0)),
        ],
        out_shape=[
            jax.ShapeDtypeStruct((m, 2), jnp.float32),
            jax.ShapeDtypeStruct((m, 2), jnp.int32),
            jax.ShapeDtypeStruct((1, 1), jnp.float32),
        ],
        scratch_shapes=[pltpu.VMEM((1, num_experts), jnp.float32)],
    )(x_flat, w1_b, b1_2d, W2)

    return scores, idx, loss[0, 0]


# speedup vs baseline: 1.0437x; 1.0437x over previous
import functools

import jax
import jax.numpy as jnp
from jax.experimental import pallas as pl
from jax.experimental.pallas import tpu as pltpu


def _gate_kernel(x_ref, w1_ref, b1_ref, w2_ref,
                 scores_ref, idx_ref, part_ref,
                 *, num_experts: int):
    x = x_ref[...]
    h = jnp.maximum(
        jax.lax.dot_general(x, w1_ref[...], (((1,), (0,)), ((), ())),
                            preferred_element_type=jnp.float32)
        + b1_ref[...],
        0.0)
    logits = jax.lax.dot_general(h, w2_ref[...], (((1,), (0,)), ((), ())),
                                 preferred_element_type=jnp.float32)

    row_max = jnp.max(logits, axis=-1, keepdims=True)
    ex = jnp.exp(logits - row_max)
    scores = ex / jnp.sum(ex, axis=-1, keepdims=True)

    iota = jax.lax.broadcasted_iota(jnp.int32, scores.shape, 1)
    m1 = jnp.max(scores, axis=-1, keepdims=True)
    a1 = jnp.min(jnp.where(scores == m1, iota, num_experts), axis=-1,
                 keepdims=True)
    masked = jnp.where(iota == a1, -jnp.inf, scores)
    m2 = jnp.max(masked, axis=-1, keepdims=True)
    a2 = jnp.min(jnp.where(masked == m2, iota, num_experts), axis=-1,
                 keepdims=True)

    t = jnp.exp(m2 - m1)
    denom = 1.0 + t
    scores_ref[...] = jnp.concatenate([1.0 / denom, t / denom], axis=-1)
    idx_ref[...] = jnp.concatenate([a1, a2], axis=-1)
    part_ref[...] = jnp.sum(scores, axis=0, keepdims=True)[None]


def _loss_kernel(part_ref, loss_ref, *, total_tokens: int, num_experts: int):
    mean = jnp.sum(part_ref[...][:, 0, :], axis=0, keepdims=True) / total_tokens
    loss_ref[...] = num_experts * jnp.sum(mean * jnp.log(mean + 1e-8),
                                          axis=-1, keepdims=True)


def kernel(x, W1, b1, W2):
    batch, seq, d_model = x.shape
    m = batch * seq
    d_hidden = W1.shape[1]
    num_experts = W2.shape[1]

    tm = min(512, m)
    nblk = m // tm
    grid = (nblk,)

    x_flat = x.reshape(m, d_model)
    b1_2d = b1.reshape(1, d_hidden)

    scores, idx, part = pl.pallas_call(
        functools.partial(_gate_kernel, num_experts=num_experts),
        grid=grid,
        in_specs=[
            pl.BlockSpec((tm, d_model), lambda i: (i, 0)),
            pl.BlockSpec((d_model, d_hidden), lambda i: (0, 0)),
            pl.BlockSpec((1, d_hidden), lambda i: (0, 0)),
            pl.BlockSpec((d_hidden, num_experts), lambda i: (0, 0)),
        ],
        out_specs=[
            pl.BlockSpec((tm, 2), lambda i: (i, 0)),
            pl.BlockSpec((tm, 2), lambda i: (i, 0)),
            pl.BlockSpec((1, 1, num_experts), lambda i: (i, 0, 0)),
        ],
        out_shape=[
            jax.ShapeDtypeStruct((m, 2), jnp.float32),
            jax.ShapeDtypeStruct((m, 2), jnp.int32),
            jax.ShapeDtypeStruct((nblk, 1, num_experts), jnp.float32),
        ],
        compiler_params=pltpu.CompilerParams(
            dimension_semantics=("parallel",)),
    )(x_flat, W1, b1_2d, W2)

    loss = pl.pallas_call(
        functools.partial(_loss_kernel, total_tokens=m,
                          num_experts=num_experts),
        out_shape=jax.ShapeDtypeStruct((1, 1), jnp.float32),
    )(part)

    return scores, idx, loss[0, 0]


# speedup vs baseline: 1.0491x; 1.0051x over previous
import functools

import jax
import jax.numpy as jnp
from jax.experimental import pallas as pl
from jax.experimental.pallas import tpu as pltpu


def _gate_kernel(x_ref, w1_ref, b1_ref, w2_ref,
                 scores_ref, idx_ref, loss_ref, acc_ref,
                 *, total_tokens: int, num_experts: int):
    i = pl.program_id(0)
    n = pl.num_programs(0)

    x = x_ref[...]
    h = jnp.maximum(
        jax.lax.dot_general(x, w1_ref[...], (((1,), (0,)), ((), ())),
                            preferred_element_type=jnp.float32)
        + b1_ref[...],
        0.0)
    logits = jax.lax.dot_general(h, w2_ref[...], (((1,), (0,)), ((), ())),
                                 preferred_element_type=jnp.float32)

    row_max = jnp.max(logits, axis=-1, keepdims=True)
    ex = jnp.exp(logits - row_max)
    scores = ex / jnp.sum(ex, axis=-1, keepdims=True)

    iota = jax.lax.broadcasted_iota(jnp.int32, scores.shape, 1)
    m1 = jnp.max(scores, axis=-1, keepdims=True)
    a1 = jnp.min(jnp.where(scores == m1, iota, num_experts), axis=-1,
                 keepdims=True)
    masked = jnp.where(iota == a1, -jnp.inf, scores)
    m2 = jnp.max(masked, axis=-1, keepdims=True)
    a2 = jnp.min(jnp.where(masked == m2, iota, num_experts), axis=-1,
                 keepdims=True)

    t = jnp.exp(m2 - m1)
    denom = 1.0 + t
    scores_ref[...] = jnp.concatenate([1.0 / denom, t / denom], axis=-1)
    idx_ref[...] = jnp.concatenate([a1, a2], axis=-1)

    part = jnp.sum(scores, axis=0, keepdims=True)
    acc_ref[...] = jnp.where(i == 0, part, acc_ref[...] + part)

    @pl.when(i == n - 1)
    def _():
        mean = acc_ref[...] / total_tokens
        loss_ref[...] = num_experts * jnp.sum(mean * jnp.log(mean + 1e-8),
                                              axis=-1, keepdims=True)


def kernel(x, W1, b1, W2):
    batch, seq, d_model = x.shape
    m = batch * seq
    d_hidden = W1.shape[1]
    num_experts = W2.shape[1]

    tm = min(512, m)
    grid = (m // tm,)

    x_flat = x.reshape(m, d_model)
    b1_2d = b1.reshape(1, d_hidden)

    scores, idx, loss = pl.pallas_call(
        functools.partial(_gate_kernel, total_tokens=m,
                          num_experts=num_experts),
        grid=grid,
        in_specs=[
            pl.BlockSpec((tm, d_model), lambda i: (i, 0)),
            pl.BlockSpec((d_model, d_hidden), lambda i: (0, 0)),
            pl.BlockSpec((1, d_hidden), lambda i: (0, 0)),
            pl.BlockSpec((d_hidden, num_experts), lambda i: (0, 0)),
        ],
        out_specs=[
            pl.BlockSpec((tm, 2), lambda i: (i, 0)),
            pl.BlockSpec((tm, 2), lambda i: (i, 0)),
            pl.BlockSpec((1, 1), lambda i: (0, 0)),
        ],
        out_shape=[
            jax.ShapeDtypeStruct((m, 2), jnp.float32),
            jax.ShapeDtypeStruct((m, 2), jnp.int32),
            jax.ShapeDtypeStruct((1, 1), jnp.float32),
        ],
        scratch_shapes=[pltpu.VMEM((1, num_experts), jnp.float32)],
    )(x_flat, W1, b1_2d, W2)

    return scores, idx, loss[0, 0]


# speedup vs baseline: 1.0526x; 1.0033x over previous
import functools

import jax
import jax.numpy as jnp
from jax.experimental import pallas as pl
from jax.experimental.pallas import tpu as pltpu


def _epilogue(i, pre, b1_ref, w2_ref, scores_ref, idx_ref, acc_ref,
              num_experts):
    h = jnp.maximum(pre + b1_ref[...], 0.0)
    logits = jax.lax.dot_general(h, w2_ref[...], (((1,), (0,)), ((), ())),
                                 preferred_element_type=jnp.float32)

    row_max = jnp.max(logits, axis=-1, keepdims=True)
    ex = jnp.exp(logits - row_max)
    scores = ex / jnp.sum(ex, axis=-1, keepdims=True)

    iota = jax.lax.broadcasted_iota(jnp.int32, scores.shape, 1)
    m1 = jnp.max(scores, axis=-1, keepdims=True)
    a1 = jnp.min(jnp.where(scores == m1, iota, num_experts), axis=-1,
                 keepdims=True)
    masked = jnp.where(iota == a1, -jnp.inf, scores)
    m2 = jnp.max(masked, axis=-1, keepdims=True)
    a2 = jnp.min(jnp.where(masked == m2, iota, num_experts), axis=-1,
                 keepdims=True)

    t = jnp.exp(m2 - m1)
    denom = 1.0 + t
    scores_ref[...] = jnp.concatenate([1.0 / denom, t / denom], axis=-1)
    idx_ref[...] = jnp.concatenate([a1, a2], axis=-1)

    part = jnp.sum(scores, axis=0, keepdims=True)
    acc_ref[...] = jnp.where(i == 0, part, acc_ref[...] + part)


def _gate_kernel(x_ref, w1_hbm, b1_ref, w2_ref,
                 scores_ref, idx_ref, loss_ref,
                 w1v_ref, acc_ref, sem_ref,
                 *, total_tokens: int, num_experts: int, nchunk: int):
    i = pl.program_id(0)
    n = pl.num_programs(0)
    d_model = w1v_ref.shape[0]
    kc = d_model // nchunk

    @pl.when(i == 0)
    def _first():
        for c in range(nchunk):
            pltpu.make_async_copy(
                w1_hbm.at[pl.ds(c * kc, kc), :],
                w1v_ref.at[pl.ds(c * kc, kc), :],
                sem_ref.at[c]).start()
        pre = None
        for c in range(nchunk):
            pltpu.make_async_copy(
                w1_hbm.at[pl.ds(c * kc, kc), :],
                w1v_ref.at[pl.ds(c * kc, kc), :],
                sem_ref.at[c]).wait()
            p = jax.lax.dot_general(
                x_ref[:, pl.ds(c * kc, kc)],
                w1v_ref[pl.ds(c * kc, kc), :],
                (((1,), (0,)), ((), ())),
                preferred_element_type=jnp.float32)
            pre = p if pre is None else pre + p
        _epilogue(i, pre, b1_ref, w2_ref, scores_ref, idx_ref, acc_ref,
                  num_experts)

    @pl.when(i > 0)
    def _rest():
        pre = jax.lax.dot_general(x_ref[...], w1v_ref[...],
                                  (((1,), (0,)), ((), ())),
                                  preferred_element_type=jnp.float32)
        _epilogue(i, pre, b1_ref, w2_ref, scores_ref, idx_ref, acc_ref,
                  num_experts)

    @pl.when(i == n - 1)
    def _loss():
        mean = acc_ref[...] / total_tokens
        loss_ref[...] = num_experts * jnp.sum(mean * jnp.log(mean + 1e-8),
                                              axis=-1, keepdims=True)


def kernel(x, W1, b1, W2):
    batch, seq, d_model = x.shape
    m = batch * seq
    d_hidden = W1.shape[1]
    num_experts = W2.shape[1]

    tm = min(512, m)
    grid = (m // tm,)

    x_flat = x.reshape(m, d_model)
    b1_2d = b1.reshape(1, d_hidden)

    scores, idx, loss = pl.pallas_call(
        functools.partial(_gate_kernel, total_tokens=m,
                          num_experts=num_experts, nchunk=4),
        grid=grid,
        in_specs=[
            pl.BlockSpec((tm, d_model), lambda i: (i, 0)),
            pl.BlockSpec(memory_space=pl.ANY),
            pl.BlockSpec((1, d_hidden), lambda i: (0, 0)),
            pl.BlockSpec((d_hidden, num_experts), lambda i: (0, 0)),
        ],
        out_specs=[
            pl.BlockSpec((tm, 2), lambda i: (i, 0)),
            pl.BlockSpec((tm, 2), lambda i: (i, 0)),
            pl.BlockSpec((1, 1), lambda i: (0, 0)),
        ],
        out_shape=[
            jax.ShapeDtypeStruct((m, 2), jnp.float32),
            jax.ShapeDtypeStruct((m, 2), jnp.int32),
            jax.ShapeDtypeStruct((1, 1), jnp.float32),
        ],
        scratch_shapes=[
            pltpu.VMEM((d_model, d_hidden), jnp.float32),
            pltpu.VMEM((1, num_experts), jnp.float32),
            pltpu.SemaphoreType.DMA((4,)),
        ],
        compiler_params=pltpu.CompilerParams(
            vmem_limit_bytes=63 * 1024 * 1024),
    )(x_flat, W1, b1_2d, W2)

    return scores, idx, loss[0, 0]
